# Initial kernel scaffold; baseline (speedup 1.0000x reference)
import jax
import jax.numpy as jnp
from jax.experimental import pallas as pl


def kernel(x, edge_index, edge_attr, batch_index, p_raw, node_table, edge_table, enc_W, enc_b, Wh1, bh1, gamma, beta, Wh2, bh2, Wp1, bp1, Wp2, bp2, mW1, mb1, mW2, mb2, mW3, mb3):
    raise NotImplementedError("write your pallas kernel here")


# speedup vs baseline: 7.3932x; 7.3932x over previous
import functools

import jax
import jax.numpy as jnp
from jax import lax
from jax.experimental import pallas as pl
from jax.experimental.pallas import tpu as pltpu
from jax.experimental.pallas import tpu_sc as plsc

_N = 10000
_E = 320000
_DIM = 128
_PD = 20
_HD = _DIM + _PD
_W = 176
_G = 64
_NV = 21
_EV = 4

_HIGH = lax.Precision.HIGHEST


def _dot(a, b):
    return lax.dot_general(a, b, (((1,), (0,)), ((), ())),
                           preferred_element_type=jnp.float32)


def _dotx(a, b):
    return lax.dot_general(a, b, (((1,), (0,)), ((), ())),
                           preferred_element_type=jnp.float32,
                           precision=_HIGH)


def _prep_body(x_ref, praw_ref, nt_ref, encW_ref, encb_ref, h_ref, p_ref):
    x = x_ref[...]
    onehot = (x == lax.broadcasted_iota(jnp.int32, (1, _NV), 1))
    h_ref[...] = _dotx(onehot.astype(jnp.float32), nt_ref[...])
    p_ref[...] = jnp.tanh(_dot(praw_ref[...], encW_ref[...]) + encb_ref[...])


_prep = pl.pallas_call(
    _prep_body,
    out_shape=[
        jax.ShapeDtypeStruct((_N, _DIM), jnp.float32),
        jax.ShapeDtypeStruct((_N, _PD), jnp.float32),
    ],
)


_WP = 128


_BN = 1000
_NB = _N // _BN


def _tblh_body(h_ref, et_ref, tblh_ref):
    h = h_ref[...]
    for a in range(_EV):
        tblh_ref[a] = jnp.maximum(h + et_ref[a:a + 1, :], 0.0)


_tblh_build = pl.pallas_call(
    _tblh_body,
    grid=(_NB,),
    in_specs=[
        pl.BlockSpec((_BN, _DIM), lambda i: (i, 0)),
        pl.BlockSpec((_EV, _DIM), lambda i: (0, 0)),
    ],
    out_specs=pl.BlockSpec((_EV, _BN, _DIM), lambda i: (0, i, 0)),
    out_shape=jax.ShapeDtypeStruct((_EV, _N, _DIM), jnp.float32),
)


def _tblp_body(p_ref, tblp_ref):
    p = p_ref[...]
    tblp_ref[...] = jnp.concatenate(
        [jnp.maximum(p, 0.0), p,
         jnp.zeros((_BN, _WP - 2 * _PD), jnp.float32)], axis=-1)


_tblp_build = pl.pallas_call(
    _tblp_body,
    grid=(_NB,),
    in_specs=[pl.BlockSpec((_BN, _PD), lambda i: (i, 0))],
    out_specs=pl.BlockSpec((_BN, _WP), lambda i: (i, 0)),
    out_shape=jax.ShapeDtypeStruct((_N, _WP), jnp.float32),
)


_NPAD = 10240
_K = 128
_NW = 32
_EPAD = 327680
_EPW = _EPAD // _NW
_NCHUNK = _EPW // _K
_RPT = _NPAD // 16
_ZR = 64


def _make_sc_agg_body(width):
    def body(tbl_hbm, idx_hbm, dst_hbm, out_hbm,
             idx_v, dst_v, rows_v, zbuf_v, acc_sh, gsem, isem):
        c = lax.axis_index("c")
        s = lax.axis_index("s")
        base = (c * 16 + s) * _EPW
        di0 = pltpu.async_copy(idx_hbm.at[pl.ds(base, _K)], idx_v.at[0], isem)
        dd0 = pltpu.async_copy(dst_hbm.at[pl.ds(base, _K)], dst_v.at[0], isem)

        zero16 = jnp.zeros((16,), jnp.float32)

        def _zrow(i, carry):
            for j in range(width // 16):
                zbuf_v[i, pl.ds(j * 16, 16)] = zero16
            return carry

        lax.fori_loop(0, _ZR, _zrow, 0)
        for r in range(_RPT // _ZR):
            pltpu.sync_copy(zbuf_v, acc_sh.at[pl.ds(s * _RPT + r * _ZR, _ZR)])
        di0.wait()
        dd0.wait()
        plsc.subcore_barrier()

        def _pair(i, carry):
            for b in range(2):
                t = 2 * i + b
                dg = pltpu.async_copy(tbl_hbm.at[idx_v.at[b]], rows_v.at[b],
                                      gsem)

                @pl.when(t > 0)
                def _scatter_prev():
                    pltpu.sync_copy(rows_v.at[1 - b],
                                    acc_sh.at[dst_v.at[1 - b]], add=True)

                @pl.when(t < _NCHUNK - 1)
                def _prefetch_next():
                    off = base + (t + 1) * _K
                    di = pltpu.async_copy(idx_hbm.at[pl.ds(off, _K)],
                                          idx_v.at[1 - b], isem)
                    dd = pltpu.async_copy(dst_hbm.at[pl.ds(off, _K)],
                                          dst_v.at[1 - b], isem)
                    di.wait()
                    dd.wait()

                dg.wait()
            return carry

        lax.fori_loop(0, _NCHUNK // 2, _pair, 0)
        pltpu.sync_copy(rows_v.at[1], acc_sh.at[dst_v.at[1]], add=True)
        plsc.subcore_barrier()
        pltpu.sync_copy(acc_sh.at[pl.ds(s * _RPT, _RPT)],
                        out_hbm.at[c, pl.ds(s * _RPT, _RPT)])

    return body


def _make_sc_agg(width):
    return pl.kernel(
        _make_sc_agg_body(width),
        out_type=jax.ShapeDtypeStruct((2, _NPAD, width), jnp.float32),
        mesh=plsc.VectorSubcoreMesh(core_axis_name="c", subcore_axis_name="s",
                                    num_cores=2, num_subcores=16),
        scratch_types=[
            pltpu.VMEM((2, _K), jnp.int32),
            pltpu.VMEM((2, _K), jnp.int32),
            pltpu.VMEM((2, _K, width), jnp.float32),
            pltpu.VMEM((_ZR, width), jnp.float32),
            pltpu.VMEM_SHARED((_NPAD, width), jnp.float32),
            pltpu.SemaphoreType.DMA,
            pltpu.SemaphoreType.DMA,
        ],
    )


@functools.cache
def _get_sc_agg_h():
    return _make_sc_agg(_DIM)


@functools.cache
def _get_sc_agg_p():
    return _make_sc_agg(_WP)


def _denseA_body(h_ref, p_ref, aggh_ref, aggp_ref, Wh1_ref, bh1_ref, Wp1_ref,
                 bp1_ref, Wp2_ref, bp2_ref, t_ref, stats_ref, pout_ref,
                 acc_ref):
    i = pl.program_id(0)
    h = h_ref[...]
    p = p_ref[...]
    ah = aggh_ref[0] + aggh_ref[1]
    ap = aggp_ref[0] + aggp_ref[1]
    z = jnp.concatenate([h + ah, p + ap[:, :_PD]], axis=-1)
    t = _dot(z, Wh1_ref[...]) + bh1_ref[...]
    t_ref[...] = t

    @pl.when(i == 0)
    def _init():
        acc_ref[...] = jnp.zeros((1, _HD), jnp.float32)

    acc_ref[...] += jnp.sum(t, axis=0, keepdims=True)

    @pl.when(i == _NB - 1)
    def _fin():
        stats_ref[...] = acc_ref[...]

    pz = p + ap[:, _PD:2 * _PD]
    q = jnp.maximum(_dot(pz, Wp1_ref[...]) + bp1_ref[...], 0.0)
    pout_ref[...] = _dot(q, Wp2_ref[...]) + bp2_ref[...] + p


_denseA = pl.pallas_call(
    _denseA_body,
    grid=(_NB,),
    in_specs=[
        pl.BlockSpec((_BN, _DIM), lambda i: (i, 0)),
        pl.BlockSpec((_BN, _PD), lambda i: (i, 0)),
        pl.BlockSpec((2, _BN, _DIM), lambda i: (0, i, 0)),
        pl.BlockSpec((2, _BN, _WP), lambda i: (0, i, 0)),
        pl.BlockSpec((_HD, _HD), lambda i: (0, 0)),
        pl.BlockSpec((1, _HD), lambda i: (0, 0)),
        pl.BlockSpec((_PD, _PD), lambda i: (0, 0)),
        pl.BlockSpec((1, _PD), lambda i: (0, 0)),
        pl.BlockSpec((_PD, _PD), lambda i: (0, 0)),
        pl.BlockSpec((1, _PD), lambda i: (0, 0)),
    ],
    out_specs=[
        pl.BlockSpec((_BN, _HD), lambda i: (i, 0)),
        pl.BlockSpec((1, _HD), lambda i: (0, 0)),
        pl.BlockSpec((_BN, _PD), lambda i: (i, 0)),
    ],
    out_shape=[
        jax.ShapeDtypeStruct((_N, _HD), jnp.float32),
        jax.ShapeDtypeStruct((1, _HD), jnp.float32),
        jax.ShapeDtypeStruct((_N, _PD), jnp.float32),
    ],
    scratch_shapes=[pltpu.VMEM((1, _HD), jnp.float32)],
)


def _denseB_body(t_ref, stats_ref, g_ref, b_ref, Wh2_ref, bh2_ref, hskip_ref,
                 hout_ref, vacc_ref):
    ph = pl.program_id(0)
    i = pl.program_id(1)
    t = t_ref[...]
    mu = stats_ref[...] * (1.0 / _N)

    @pl.when(ph == 0)
    def _accum_var():
        @pl.when(i == 0)
        def _init():
            vacc_ref[...] = jnp.zeros((1, _HD), jnp.float32)

        d = t - mu
        vacc_ref[...] += jnp.sum(d * d, axis=0, keepdims=True)

    @pl.when(ph == 1)
    def _normalize():
        var = vacc_ref[...] * (1.0 / _N)
        tn = jnp.maximum((t - mu) / jnp.sqrt(var + 1e-5) * g_ref[...]
                         + b_ref[...], 0.0)
        hout_ref[...] = (_dot(tn, Wh2_ref[...]) + bh2_ref[...]
                         + hskip_ref[...])


_denseB = pl.pallas_call(
    _denseB_body,
    grid=(2, _NB),
    in_specs=[
        pl.BlockSpec((_BN, _HD), lambda ph, i: (i, 0)),
        pl.BlockSpec((1, _HD), lambda ph, i: (0, 0)),
        pl.BlockSpec((1, _HD), lambda ph, i: (0, 0)),
        pl.BlockSpec((1, _HD), lambda ph, i: (0, 0)),
        pl.BlockSpec((_HD, _DIM), lambda ph, i: (0, 0)),
        pl.BlockSpec((1, _DIM), lambda ph, i: (0, 0)),
        pl.BlockSpec((_BN, _DIM), lambda ph, i: (i, 0)),
    ],
    out_specs=pl.BlockSpec((_BN, _DIM), lambda ph, i: (i, 0)),
    out_shape=jax.ShapeDtypeStruct((_N, _DIM), jnp.float32),
    scratch_shapes=[pltpu.VMEM((1, _HD), jnp.float32)],
)


def _pool_body(h_ref, p_ref, biT_ref, mW1_ref, mb1_ref, mW2_ref, mb2_ref,
               mW3_ref, mb3_ref, out_ref):
    hp1 = jnp.concatenate(
        [h_ref[...], p_ref[...], jnp.ones((_N, 1), jnp.float32)], axis=-1)
    onehotT = (lax.broadcasted_iota(jnp.int32, (_G, 1), 0) == biT_ref[...])
    sums = _dotx(onehotT.astype(jnp.float32), hp1)
    cnt = jnp.maximum(sums[:, _HD:_HD + 1], 1.0)
    g = sums[:, :_HD] / cnt
    o = jnp.maximum(_dot(g, mW1_ref[...]) + mb1_ref[...], 0.0)
    o = jnp.maximum(_dot(o, mW2_ref[...]) + mb2_ref[...], 0.0)
    out_ref[...] = _dot(o, mW3_ref[...]) + mb3_ref[...]


_pool = pl.pallas_call(
    _pool_body,
    out_shape=jax.ShapeDtypeStruct((_G, 1), jnp.float32),
)


def kernel(x, edge_index, edge_attr, batch_index, p_raw, node_table,
           edge_table, enc_W, enc_b, Wh1, bh1, gamma, beta, Wh2, bh2, Wp1,
           bp1, Wp2, bp2, mW1, mb1, mW2, mb2, mW3, mb3):
    x2 = x.reshape(_N, 1).astype(jnp.int32)
    biT = batch_index.reshape(1, _N).astype(jnp.int32)
    src = edge_index[0].astype(jnp.int32)
    npadE = _EPAD - _E
    pad_src = jnp.arange(npadE, dtype=jnp.int32) % _N
    pad_dst = _N + (jnp.arange(npadE, dtype=jnp.int32) % (_NPAD - _N))
    src_p = jnp.concatenate([src, pad_src])
    eidx = jnp.concatenate(
        [edge_attr.astype(jnp.int32) * _N + src, pad_src])
    dst = jnp.concatenate([edge_index[1].astype(jnp.int32), pad_dst])

    h, p = _prep(x2, p_raw, node_table, enc_W, enc_b.reshape(1, _PD))

    for l in range(3):
        tblp = _tblp_build(p)
        aggp = _get_sc_agg_p()(tblp, src_p, dst)
        tblh = _tblh_build(h, edge_table)
        aggh = _get_sc_agg_h()(tblh.reshape(_EV * _N, _DIM), eidx, dst)
        t, stats, p = _denseA(h, p, aggh, aggp, Wh1[l],
                              bh1[l].reshape(1, _HD),
                              Wp1[l], bp1[l].reshape(1, _PD), Wp2[l],
                              bp2[l].reshape(1, _PD))
        h = _denseB(t, stats, gamma[l].reshape(1, _HD),
                    beta[l].reshape(1, _HD), Wh2[l], bh2[l].reshape(1, _DIM),
                    h)

    return _pool(h, p, biT, mW1, mb1.reshape(1, _HD // 2), mW2,
                 mb2.reshape(1, _HD // 4), mW3, mb3.reshape(1, 1))


# speedup vs baseline: 7.4043x; 1.0015x over previous
import functools

import jax
import jax.numpy as jnp
from jax import lax
from jax.experimental import pallas as pl
from jax.experimental.pallas import tpu as pltpu
from jax.experimental.pallas import tpu_sc as plsc

_N = 10000
_E = 320000
_DIM = 128
_PD = 20
_HD = _DIM + _PD
_W = 176
_G = 64
_NV = 21
_EV = 4

_HIGH = lax.Precision.HIGHEST


def _dot(a, b):
    return lax.dot_general(a, b, (((1,), (0,)), ((), ())),
                           preferred_element_type=jnp.float32)


def _dotx(a, b):
    return lax.dot_general(a, b, (((1,), (0,)), ((), ())),
                           preferred_element_type=jnp.float32,
                           precision=_HIGH)


def _prep_body(x_ref, praw_ref, nt_ref, encW_ref, encb_ref, h_ref, p_ref):
    x = x_ref[...]
    onehot = (x == lax.broadcasted_iota(jnp.int32, (1, _NV), 1))
    h_ref[...] = _dotx(onehot.astype(jnp.float32), nt_ref[...])
    p_ref[...] = jnp.tanh(_dot(praw_ref[...], encW_ref[...]) + encb_ref[...])


_prep = pl.pallas_call(
    _prep_body,
    out_shape=[
        jax.ShapeDtypeStruct((_N, _DIM), jnp.float32),
        jax.ShapeDtypeStruct((_N, _PD), jnp.float32),
    ],
)


_WP = 128


_BN = 1000
_NB = _N // _BN


def _tblh_body(h_ref, et_ref, tblh_ref):
    h = h_ref[...]
    for a in range(_EV):
        tblh_ref[a] = jnp.maximum(h + et_ref[a:a + 1, :], 0.0)


_tblh_build = pl.pallas_call(
    _tblh_body,
    grid=(_NB,),
    in_specs=[
        pl.BlockSpec((_BN, _DIM), lambda i: (i, 0)),
        pl.BlockSpec((_EV, _DIM), lambda i: (0, 0)),
    ],
    out_specs=pl.BlockSpec((_EV, _BN, _DIM), lambda i: (0, i, 0)),
    out_shape=jax.ShapeDtypeStruct((_EV, _N, _DIM), jnp.float32),
)


def _tblp_body(p_ref, tblp_ref):
    p = p_ref[...]
    tblp_ref[...] = jnp.concatenate(
        [jnp.maximum(p, 0.0), p,
         jnp.zeros((_BN, _WP - 2 * _PD), jnp.float32)], axis=-1)


_tblp_build = pl.pallas_call(
    _tblp_body,
    grid=(_NB,),
    in_specs=[pl.BlockSpec((_BN, _PD), lambda i: (i, 0))],
    out_specs=pl.BlockSpec((_BN, _WP), lambda i: (i, 0)),
    out_shape=jax.ShapeDtypeStruct((_N, _WP), jnp.float32),
)


_NPAD = 10240
_K = 128
_NW = 32
_EPAD = 327680
_EPW = _EPAD // _NW
_NCHUNK = _EPW // _K
_RPT = _NPAD // 16
_ZR = 64


def _make_sc_agg_body(width):
    def body(tbl_hbm, idx_hbm, dst_hbm, out_hbm,
             idx_v, dst_v, rows_v, zbuf_v, acc_sh, gsem, isem):
        c = lax.axis_index("c")
        s = lax.axis_index("s")
        base = (c * 16 + s) * _EPW
        di0 = pltpu.async_copy(idx_hbm.at[pl.ds(base, _K)], idx_v.at[0], isem)
        dd0 = pltpu.async_copy(dst_hbm.at[pl.ds(base, _K)], dst_v.at[0], isem)

        zero16 = jnp.zeros((16,), jnp.float32)

        def _zrow(i, carry):
            for j in range(width // 16):
                zbuf_v[i, pl.ds(j * 16, 16)] = zero16
            return carry

        lax.fori_loop(0, _ZR, _zrow, 0)
        for r in range(_RPT // _ZR):
            pltpu.sync_copy(zbuf_v, acc_sh.at[pl.ds(s * _RPT + r * _ZR, _ZR)])
        di0.wait()
        dd0.wait()
        plsc.subcore_barrier()

        def _pair(i, carry):
            for b in range(2):
                t = 2 * i + b
                dg = pltpu.async_copy(tbl_hbm.at[idx_v.at[b]], rows_v.at[b],
                                      gsem)

                @pl.when(t > 0)
                def _scatter_prev():
                    pltpu.sync_copy(rows_v.at[1 - b],
                                    acc_sh.at[dst_v.at[1 - b]], add=True)

                @pl.when(t < _NCHUNK - 1)
                def _prefetch_next():
                    off = base + (t + 1) * _K
                    di = pltpu.async_copy(idx_hbm.at[pl.ds(off, _K)],
                                          idx_v.at[1 - b], isem)
                    dd = pltpu.async_copy(dst_hbm.at[pl.ds(off, _K)],
                                          dst_v.at[1 - b], isem)
                    di.wait()
                    dd.wait()

                dg.wait()
            return carry

        lax.fori_loop(0, _NCHUNK // 2, _pair, 0)
        pltpu.sync_copy(rows_v.at[1], acc_sh.at[dst_v.at[1]], add=True)
        plsc.subcore_barrier()
        pltpu.sync_copy(acc_sh.at[pl.ds(s * _RPT, _RPT)],
                        out_hbm.at[c, pl.ds(s * _RPT, _RPT)])

    return body


def _make_sc_agg(width):
    return pl.kernel(
        _make_sc_agg_body(width),
        out_type=jax.ShapeDtypeStruct((2, _NPAD, width), jnp.float32),
        mesh=plsc.VectorSubcoreMesh(core_axis_name="c", subcore_axis_name="s",
                                    num_cores=2, num_subcores=16),
        scratch_types=[
            pltpu.VMEM((2, _K), jnp.int32),
            pltpu.VMEM((2, _K), jnp.int32),
            pltpu.VMEM((2, _K, width), jnp.float32),
            pltpu.VMEM((_ZR, width), jnp.float32),
            pltpu.VMEM_SHARED((_NPAD, width), jnp.float32),
            pltpu.SemaphoreType.DMA,
            pltpu.SemaphoreType.DMA,
        ],
    )


@functools.cache
def _get_sc_agg_h():
    return _make_sc_agg(_DIM)


@functools.cache
def _get_sc_agg_p():
    return _make_sc_agg(_WP)


def _denseA_body(h_ref, p_ref, aggh_ref, aggp_ref, Wh1_ref, bh1_ref, Wp1_ref,
                 bp1_ref, Wp2_ref, bp2_ref, t_ref, stats_ref, pout_ref,
                 acc_ref):
    i = pl.program_id(0)
    h = h_ref[...]
    p = p_ref[...]
    ah = aggh_ref[0] + aggh_ref[1]
    ap = aggp_ref[0] + aggp_ref[1]
    z = jnp.concatenate([h + ah, p + ap[:, :_PD]], axis=-1)
    t = _dot(z, Wh1_ref[...]) + bh1_ref[...]
    t_ref[...] = t

    @pl.when(i == 0)
    def _init():
        acc_ref[...] = jnp.zeros((1, _HD), jnp.float32)

    acc_ref[...] += jnp.sum(t, axis=0, keepdims=True)

    @pl.when(i == _NB - 1)
    def _fin():
        stats_ref[...] = acc_ref[...]

    pz = p + ap[:, _PD:2 * _PD]
    q = jnp.maximum(_dot(pz, Wp1_ref[...]) + bp1_ref[...], 0.0)
    pout_ref[...] = _dot(q, Wp2_ref[...]) + bp2_ref[...] + p


_denseA = pl.pallas_call(
    _denseA_body,
    grid=(_NB,),
    in_specs=[
        pl.BlockSpec((_BN, _DIM), lambda i: (i, 0)),
        pl.BlockSpec((_BN, _PD), lambda i: (i, 0)),
        pl.BlockSpec((2, _BN, _DIM), lambda i: (0, i, 0)),
        pl.BlockSpec((2, _BN, _WP), lambda i: (0, i, 0)),
        pl.BlockSpec((_HD, _HD), lambda i: (0, 0)),
        pl.BlockSpec((1, _HD), lambda i: (0, 0)),
        pl.BlockSpec((_PD, _PD), lambda i: (0, 0)),
        pl.BlockSpec((1, _PD), lambda i: (0, 0)),
        pl.BlockSpec((_PD, _PD), lambda i: (0, 0)),
        pl.BlockSpec((1, _PD), lambda i: (0, 0)),
    ],
    out_specs=[
        pl.BlockSpec((_BN, _HD), lambda i: (i, 0)),
        pl.BlockSpec((1, _HD), lambda i: (0, 0)),
        pl.BlockSpec((_BN, _PD), lambda i: (i, 0)),
    ],
    out_shape=[
        jax.ShapeDtypeStruct((_N, _HD), jnp.float32),
        jax.ShapeDtypeStruct((1, _HD), jnp.float32),
        jax.ShapeDtypeStruct((_N, _PD), jnp.float32),
    ],
    scratch_shapes=[pltpu.VMEM((1, _HD), jnp.float32)],
)


def _denseB_body(t_ref, stats_ref, g_ref, b_ref, Wh2_ref, bh2_ref, hskip_ref,
                 hout_ref, vacc_ref):
    ph = pl.program_id(0)
    i = pl.program_id(1)
    t = t_ref[...]
    mu = stats_ref[...] * (1.0 / _N)

    @pl.when(ph == 0)
    def _accum_var():
        @pl.when(i == 0)
        def _init():
            vacc_ref[...] = jnp.zeros((1, _HD), jnp.float32)

        d = t - mu
        vacc_ref[...] += jnp.sum(d * d, axis=0, keepdims=True)

    @pl.when(ph == 1)
    def _normalize():
        var = vacc_ref[...] * (1.0 / _N)
        tn = jnp.maximum((t - mu) / jnp.sqrt(var + 1e-5) * g_ref[...]
                         + b_ref[...], 0.0)
        hout_ref[...] = (_dot(tn, Wh2_ref[...]) + bh2_ref[...]
                         + hskip_ref[...])


_denseB = pl.pallas_call(
    _denseB_body,
    grid=(2, _NB),
    in_specs=[
        pl.BlockSpec((_BN, _HD), lambda ph, i: (i, 0)),
        pl.BlockSpec((1, _HD), lambda ph, i: (0, 0)),
        pl.BlockSpec((1, _HD), lambda ph, i: (0, 0)),
        pl.BlockSpec((1, _HD), lambda ph, i: (0, 0)),
        pl.BlockSpec((_HD, _DIM), lambda ph, i: (0, 0)),
        pl.BlockSpec((1, _DIM), lambda ph, i: (0, 0)),
        pl.BlockSpec((_BN, _DIM), lambda ph, i: (i, 0)),
    ],
    out_specs=pl.BlockSpec((_BN, _DIM), lambda ph, i: (i, 0)),
    out_shape=jax.ShapeDtypeStruct((_N, _DIM), jnp.float32),
    scratch_shapes=[pltpu.VMEM((1, _HD), jnp.float32)],
)


def _pool_body(h_ref, p_ref, biT_ref, mW1_ref, mb1_ref, mW2_ref, mb2_ref,
               mW3_ref, mb3_ref, out_ref):
    hp1 = jnp.concatenate(
        [h_ref[...], p_ref[...], jnp.ones((_N, 1), jnp.float32)], axis=-1)
    onehotT = (lax.broadcasted_iota(jnp.int32, (_G, 1), 0) == biT_ref[...])
    sums = _dotx(onehotT.astype(jnp.float32), hp1)
    cnt = jnp.maximum(sums[:, _HD:_HD + 1], 1.0)
    g = sums[:, :_HD] / cnt
    o = jnp.maximum(_dot(g, mW1_ref[...]) + mb1_ref[...], 0.0)
    o = jnp.maximum(_dot(o, mW2_ref[...]) + mb2_ref[...], 0.0)
    out_ref[...] = _dot(o, mW3_ref[...]) + mb3_ref[...]


_pool = pl.pallas_call(
    _pool_body,
    out_shape=jax.ShapeDtypeStruct((_G, 1), jnp.float32),
)


def kernel(x, edge_index, edge_attr, batch_index, p_raw, node_table,
           edge_table, enc_W, enc_b, Wh1, bh1, gamma, beta, Wh2, bh2, Wp1,
           bp1, Wp2, bp2, mW1, mb1, mW2, mb2, mW3, mb3):
    x2 = x.reshape(_N, 1).astype(jnp.int32)
    biT = batch_index.reshape(1, _N).astype(jnp.int32)
    src = edge_index[0].astype(jnp.int32)
    npadE = _EPAD - _E
    pad_src = jnp.arange(npadE, dtype=jnp.int32) % _N
    pad_dst = _N + (jnp.arange(npadE, dtype=jnp.int32) % (_NPAD - _N))
    src_p = jnp.concatenate([src, pad_src])
    eidx = jnp.concatenate(
        [edge_attr.astype(jnp.int32) * _N + src, pad_src])
    dst = jnp.concatenate([edge_index[1].astype(jnp.int32), pad_dst])

    h, p = _prep(x2, p_raw, node_table, enc_W, enc_b.reshape(1, _PD))

    t = stats = hskip = None
    for l in range(3):
        tblp = _tblp_build(p)
        aggp = _get_sc_agg_p()(tblp, src_p, dst)
        if l > 0:
            h = _denseB(t, stats, gamma[l - 1].reshape(1, _HD),
                        beta[l - 1].reshape(1, _HD), Wh2[l - 1],
                        bh2[l - 1].reshape(1, _DIM), hskip)
        tblh = _tblh_build(h, edge_table)
        aggh = _get_sc_agg_h()(tblh.reshape(_EV * _N, _DIM), eidx, dst)
        hskip = h
        t, stats, p = _denseA(h, p, aggh, aggp, Wh1[l],
                              bh1[l].reshape(1, _HD),
                              Wp1[l], bp1[l].reshape(1, _PD), Wp2[l],
                              bp2[l].reshape(1, _PD))
    h = _denseB(t, stats, gamma[2].reshape(1, _HD), beta[2].reshape(1, _HD),
                Wh2[2], bh2[2].reshape(1, _DIM), hskip)

    return _pool(h, p, biT, mW1, mb1.reshape(1, _HD // 2), mW2,
                 mb2.reshape(1, _HD // 4), mW3, mb3.reshape(1, 1))
